# baseline (speedup 1.0000x reference)
import functools

import jax
import jax.numpy as jnp
from jax.experimental import pallas as pl
from jax.experimental.pallas import tpu as pltpu


def _fused_kernel(x_ref, L_ref, wc_ref, bc_ref, wfc_ref, bfc_ref, out_ref,
                  *, K, G):
    L = L_ref[...]
    t_m2 = x_ref[...]
    dot = functools.partial(
        jnp.dot, preferred_element_type=jnp.float32,
        precision=jax.lax.Precision.DEFAULT)
    t_m1 = dot(t_m2, L)

    acc = [wc_ref[0, g] * t_m2 + wc_ref[1, g] * t_m1 for g in range(G)]
    for k in range(2, K):
        t = 2.0 * dot(t_m1, L) - t_m2
        for g in range(G):
            acc[g] = acc[g] + wc_ref[k, g] * t
        t_m2, t_m1 = t_m1, t

    logits = bfc_ref[...]
    for g in range(G):
        h_g = jnp.maximum(acc[g] + bc_ref[g], 0.0)
        logits = logits + dot(h_g, wfc_ref[g])

    m = jnp.max(logits, axis=1, keepdims=True)
    s = logits - m
    out_ref[...] = s - jnp.log(jnp.sum(jnp.exp(s), axis=1, keepdims=True))


def kernel(x, L, W_cheb, b_cheb, W_fc, b_fc):
    B, N, F_IN = x.shape
    K, _, G = W_cheb.shape
    D = W_fc.shape[1]
    xt = x.reshape(B, N)
    wc = W_cheb.reshape(K, G)
    wfc = W_fc.reshape(N, G, D).transpose(1, 0, 2)

    cores = 2
    bb = B // cores
    fn = pl.pallas_call(
        functools.partial(_fused_kernel, K=K, G=G),
        grid=(cores,),
        out_shape=jax.ShapeDtypeStruct((B, D), jnp.float32),
        in_specs=[
            pl.BlockSpec((bb, N), lambda i: (i, 0)),
            pl.BlockSpec((N, N), lambda i: (0, 0)),
            pl.BlockSpec(memory_space=pltpu.SMEM),
            pl.BlockSpec(memory_space=pltpu.SMEM),
            pl.BlockSpec((G, N, D), lambda i: (0, 0, 0)),
            pl.BlockSpec((1, D), lambda i: (0, 0)),
        ],
        out_specs=pl.BlockSpec((bb, D), lambda i: (i, 0)),
        compiler_params=pltpu.CompilerParams(
            dimension_semantics=("parallel",)),
    )
    return fn(xt, L, wc, b_cheb, wfc, b_fc.reshape(1, D))


# speedup vs baseline: 1.0398x; 1.0398x over previous
import functools

import jax
import jax.numpy as jnp
from jax.experimental import pallas as pl
from jax.experimental.pallas import tpu as pltpu


def _fused_kernel(x_ref, L_ref, wc_ref, bc_ref, wfc_ref, bfc_ref, out_ref,
                  *, K, G):
    Lb = L_ref[...].astype(jnp.bfloat16)
    t_m2 = x_ref[...]
    dot = functools.partial(jnp.dot, preferred_element_type=jnp.float32)
    t_m1 = dot(t_m2.astype(jnp.bfloat16), Lb)

    acc = [wc_ref[0, g] * t_m2 + wc_ref[1, g] * t_m1 for g in range(G)]
    for k in range(2, K):
        t = 2.0 * dot(t_m1.astype(jnp.bfloat16), Lb) - t_m2
        for g in range(G):
            acc[g] = acc[g] + wc_ref[k, g] * t
        t_m2, t_m1 = t_m1, t

    logits = bfc_ref[...]
    for g in range(G):
        h_g = jnp.maximum(acc[g] + bc_ref[g], 0.0)
        logits = logits + dot(h_g, wfc_ref[g])

    m = jnp.max(logits, axis=1, keepdims=True)
    s = logits - m
    out_ref[...] = s - jnp.log(jnp.sum(jnp.exp(s), axis=1, keepdims=True))


def kernel(x, L, W_cheb, b_cheb, W_fc, b_fc):
    B, N, F_IN = x.shape
    K, _, G = W_cheb.shape
    D = W_fc.shape[1]
    xt = x.reshape(B, N)
    wc = W_cheb.reshape(K, G)
    wfc = W_fc.reshape(N, G, D).transpose(1, 0, 2)

    fn = pl.pallas_call(
        functools.partial(_fused_kernel, K=K, G=G),
        out_shape=jax.ShapeDtypeStruct((B, D), jnp.float32),
        in_specs=[
            pl.BlockSpec(memory_space=pltpu.VMEM),
            pl.BlockSpec(memory_space=pltpu.VMEM),
            pl.BlockSpec(memory_space=pltpu.SMEM),
            pl.BlockSpec(memory_space=pltpu.SMEM),
            pl.BlockSpec(memory_space=pltpu.VMEM),
            pl.BlockSpec(memory_space=pltpu.VMEM),
        ],
        out_specs=pl.BlockSpec(memory_space=pltpu.VMEM),
    )
    return fn(xt, L, wc, b_cheb, wfc, b_fc.reshape(1, D))
